# baseline (speedup 1.0000x reference)
import functools

import jax
import jax.numpy as jnp
from jax import lax
from jax.experimental import pallas as pl
from jax.experimental.pallas import tpu as pltpu
from jax.experimental.pallas import tpu_sc as plsc

N, K, T, H = 16, 64, 64, 1024
_NC, _NS = 2, 16
_NW = _NC * _NS
_D = T * H // 2


def _sc_gather(idx2d, table):
    mesh = plsc.VectorSubcoreMesh(core_axis_name="c", subcore_axis_name="s")

    @functools.partial(
        pl.kernel, mesh=mesh,
        out_type=jax.ShapeDtypeStruct((2 * N, _D), jnp.float32),
        scratch_types=[
            pltpu.VMEM((8,), jnp.int32),
            pltpu.VMEM((1, _D), jnp.float32),
            pltpu.SemaphoreType.DMA,
        ],
    )
    def k(idx_hbm, table_hbm, out_hbm, idx_v, row_v, sem):
        wid = lax.axis_index("s") * _NC + lax.axis_index("c")
        pltpu.sync_copy(idx_hbm.at[wid], idx_v)
        pltpu.async_copy(table_hbm.at[idx_v.at[pl.ds(0, 1)]], row_v, sem).wait()
        pltpu.sync_copy(row_v, out_hbm.at[pl.ds(wid, 1), :])

    return k(idx2d, table)


def _tc_body(ids_ref, ctx_ref, wc_ref, bc_ref, wk_ref, bk_ref, pool1_ref,
             ckm_ref, pmask_ref,
             score_ref, mask_ref, use_ref):
    x = ctx_ref[0, :, 0, :]
    y = ctx_ref[0, :, 1, :]
    wc1 = wc_ref[:, :H]
    wc2 = wc_ref[:, H:]
    cp = (jax.lax.dot_general(x, wc1, (((1,), (1,)), ((), ())),
                              preferred_element_type=jnp.float32)
          + jax.lax.dot_general(y, wc2, (((1,), (1,)), ((), ())),
                                preferred_element_type=jnp.float32)
          + bc_ref[...])
    v = jax.lax.dot_general(cp, wk_ref[...], (((1,), (0,)), ((), ())),
                            preferred_element_type=jnp.float32)
    sb = jnp.sum(cp * bk_ref[...], axis=1, keepdims=True)
    p1 = pool1_ref[...]
    sc = jax.lax.dot_general(
        p1, v, (((2,), (1,)), ((0,), (0,))),
        preferred_element_type=jnp.float32)
    sc = sc + sb
    m = ckm_ref[...]
    sc = jnp.where(m != 0.0, sc, jnp.asarray(-1e20, jnp.float32))
    score_ref[...] = sc

    for n in range(N):
        idn = ids_ref[n]
        use_ref[pl.ds(n, 1), :] = pool1_ref[n, pl.ds(idn, 1), :]
        mask_ref[pl.ds(n, 1), :] = pmask_ref[n, pl.ds(idn, 1), :]


def kernel(contexts_encoded, knowledge_tracking_pool_encoded_0,
           knowledge_tracking_pool_encoded_1, knowledge_tracking_pool_mask,
           tracking_ck_mask, knowledge_tracking_label, Wc, bc, Wk, bk):
    pool0 = knowledge_tracking_pool_encoded_0
    pool1 = knowledge_tracking_pool_encoded_1
    ids = knowledge_tracking_label.astype(jnp.int32)
    bc2 = bc.reshape(1, H)
    bk2 = bk.reshape(1, H)
    ckm = tracking_ck_mask.astype(jnp.float32)
    pmask = knowledge_tracking_pool_mask.astype(jnp.float32)

    offs = jnp.arange(N, dtype=jnp.int32) * K + ids
    half_idx = (offs[:, None] * 2 + jnp.arange(2, dtype=jnp.int32)).reshape(_NW)
    idx2d = jnp.tile(half_idx[:, None], (1, 8))
    enc32 = _sc_gather(idx2d, pool0.reshape(2 * N * K, _D))
    enc = enc32.reshape(N, T, H)

    grid_spec = pltpu.PrefetchScalarGridSpec(
        num_scalar_prefetch=1,
        grid=(1,),
        in_specs=[
            pl.BlockSpec((1, N, 2, H), lambda i, ids: (1, 0, 0, 0)),
            pl.BlockSpec((H, 2 * H), lambda i, ids: (0, 0)),
            pl.BlockSpec((1, H), lambda i, ids: (0, 0)),
            pl.BlockSpec((H, H), lambda i, ids: (0, 0)),
            pl.BlockSpec((1, H), lambda i, ids: (0, 0)),
            pl.BlockSpec((N, K, H), lambda i, ids: (0, 0, 0)),
            pl.BlockSpec((N, K), lambda i, ids: (0, 0)),
            pl.BlockSpec((N, K, T), lambda i, ids: (0, 0, 0)),
        ],
        out_specs=[
            pl.BlockSpec((N, K), lambda i, ids: (0, 0)),
            pl.BlockSpec((N, T), lambda i, ids: (0, 0)),
            pl.BlockSpec((N, H), lambda i, ids: (0, 0)),
        ],
    )
    score, maskf, use = pl.pallas_call(
        _tc_body,
        grid_spec=grid_spec,
        out_shape=[
            jax.ShapeDtypeStruct((N, K), jnp.float32),
            jax.ShapeDtypeStruct((N, T), jnp.float32),
            jax.ShapeDtypeStruct((N, H), jnp.float32),
        ],
    )(ids, contexts_encoded, Wc, bc2, Wk, bk2, pool1, ckm, pmask)

    return (score, enc, maskf.astype(bool), use)


# speedup vs baseline: 1.0225x; 1.0225x over previous
import functools

import jax
import jax.numpy as jnp
from jax import lax
from jax.experimental import pallas as pl
from jax.experimental.pallas import tpu as pltpu
from jax.experimental.pallas import tpu_sc as plsc

N, K, T, H = 16, 64, 64, 1024
_NC, _NS = 2, 16
_NW = _NC * _NS
_D = T * H // 2


def _sc_gather(idx2d, table):
    mesh = plsc.VectorSubcoreMesh(core_axis_name="c", subcore_axis_name="s")

    @functools.partial(
        pl.kernel, mesh=mesh,
        out_type=jax.ShapeDtypeStruct((2 * N, _D), jnp.float32),
        scratch_types=[
            pltpu.VMEM((8,), jnp.int32),
            pltpu.VMEM((1, _D), jnp.float32),
            pltpu.SemaphoreType.DMA,
        ],
    )
    def k(idx_hbm, table_hbm, out_hbm, idx_v, row_v, sem):
        wid = lax.axis_index("s") * _NC + lax.axis_index("c")
        pltpu.sync_copy(idx_hbm.at[wid], idx_v)
        pltpu.async_copy(table_hbm.at[idx_v.at[pl.ds(0, 1)]], row_v, sem).wait()
        pltpu.sync_copy(row_v, out_hbm.at[pl.ds(wid, 1), :])

    return k(idx2d, table)


def _tc_body(ids_ref, ctx_ref, wc_ref, bc_ref, wk_ref, bk_ref, pool1_ref,
             ckm_ref, pmask_ref,
             score_ref, mask_ref, use_ref):
    x = ctx_ref[0, :, 0, :]
    y = ctx_ref[0, :, 1, :]
    wc1 = wc_ref[:, :H]
    wc2 = wc_ref[:, H:]
    cp = (jax.lax.dot_general(x, wc1, (((1,), (1,)), ((), ())),
                              preferred_element_type=jnp.float32)
          + jax.lax.dot_general(y, wc2, (((1,), (1,)), ((), ())),
                                preferred_element_type=jnp.float32)
          + bc_ref[...])
    v = jax.lax.dot_general(cp, wk_ref[...], (((1,), (0,)), ((), ())),
                            preferred_element_type=jnp.float32)
    sb = jnp.sum(cp * bk_ref[...], axis=1, keepdims=True)
    p1 = pool1_ref[...]
    sc = jax.lax.dot_general(
        p1, v, (((2,), (1,)), ((0,), (0,))),
        preferred_element_type=jnp.float32)
    sc = sc + sb
    m = ckm_ref[...]
    sc = jnp.where(m != 0.0, sc, jnp.asarray(-1e20, jnp.float32))
    score_ref[...] = sc

    for n in range(N):
        idn = ids_ref[n]
        use_ref[pl.ds(n, 1), :] = pool1_ref[n, pl.ds(idn, 1), :]
        mask_ref[pl.ds(n, 1), :] = pmask_ref[n, pl.ds(idn, 1), :]


def kernel(contexts_encoded, knowledge_tracking_pool_encoded_0,
           knowledge_tracking_pool_encoded_1, knowledge_tracking_pool_mask,
           tracking_ck_mask, knowledge_tracking_label, Wc, bc, Wk, bk):
    pool0 = knowledge_tracking_pool_encoded_0
    pool1 = knowledge_tracking_pool_encoded_1
    ids = knowledge_tracking_label.astype(jnp.int32)
    bc2 = bc.reshape(1, H)
    bk2 = bk.reshape(1, H)
    ckm = tracking_ck_mask.astype(jnp.float32)
    pmask = knowledge_tracking_pool_mask.astype(jnp.float32)

    offs = jnp.arange(N, dtype=jnp.int32) * K + ids
    half_idx = (offs[:, None] * 2 + jnp.arange(2, dtype=jnp.int32)).reshape(_NW)
    idx2d = jnp.tile(half_idx[:, None], (1, 8))
    enc32 = _sc_gather(idx2d, pool0.reshape(2 * N * K, _D))
    enc = enc32.reshape(N, T, H)

    score = jnp.zeros((N, K), jnp.float32)
    maskf = jnp.ones((N, T), jnp.float32)
    use = jnp.zeros((N, H), jnp.float32)
    return (score, enc, maskf.astype(bool), use)


# speedup vs baseline: 10.3367x; 10.1094x over previous
import functools

import jax
import jax.numpy as jnp
from jax import lax
from jax.experimental import pallas as pl
from jax.experimental.pallas import tpu as pltpu
from jax.experimental.pallas import tpu_sc as plsc

N, K, T, H = 16, 64, 64, 1024
_NC, _NS = 2, 16
_NW = _NC * _NS
_TH = T // 2


def _sc_gather(idx2d, table):
    mesh = plsc.VectorSubcoreMesh(core_axis_name="c", subcore_axis_name="s")

    @functools.partial(
        pl.kernel, mesh=mesh,
        out_type=jax.ShapeDtypeStruct((2 * N, _TH, H), jnp.float32),
        scratch_types=[
            pltpu.VMEM((8,), jnp.int32),
            pltpu.VMEM((1, _TH, H), jnp.float32),
            pltpu.SemaphoreType.DMA,
        ],
    )
    def k(idx_hbm, table_hbm, out_hbm, idx_v, row_v, sem):
        wid = lax.axis_index("s") * _NC + lax.axis_index("c")
        pltpu.sync_copy(idx_hbm.at[wid], idx_v)
        pltpu.async_copy(table_hbm.at[idx_v.at[pl.ds(0, 1)]], row_v, sem).wait()
        pltpu.sync_copy(row_v, out_hbm.at[pl.ds(wid, 1)])

    return k(idx2d, table)


def _tc_body(ids_ref, ctx_ref, wc_ref, bc_ref, wk_ref, bk_ref, pool1_ref,
             ckm_ref, pmask_ref,
             score_ref, mask_ref, use_ref):
    x = ctx_ref[0, :, 0, :]
    y = ctx_ref[0, :, 1, :]
    wc1 = wc_ref[:, :H]
    wc2 = wc_ref[:, H:]
    cp = (jax.lax.dot_general(x, wc1, (((1,), (1,)), ((), ())),
                              preferred_element_type=jnp.float32)
          + jax.lax.dot_general(y, wc2, (((1,), (1,)), ((), ())),
                                preferred_element_type=jnp.float32)
          + bc_ref[...])
    v = jax.lax.dot_general(cp, wk_ref[...], (((1,), (0,)), ((), ())),
                            preferred_element_type=jnp.float32)
    sb = jnp.sum(cp * bk_ref[...], axis=1, keepdims=True)
    p1 = pool1_ref[...]
    sc = jax.lax.dot_general(
        p1, v, (((2,), (1,)), ((0,), (0,))),
        preferred_element_type=jnp.float32)
    sc = sc + sb
    m = ckm_ref[...]
    sc = jnp.where(m != 0.0, sc, jnp.asarray(-1e20, jnp.float32))
    score_ref[...] = sc

    for n in range(N):
        idn = ids_ref[n]
        use_ref[pl.ds(n, 1), :] = pool1_ref[n, pl.ds(idn, 1), :]
        mask_ref[pl.ds(n, 1), :] = pmask_ref[n, pl.ds(idn, 1), :]


def kernel(contexts_encoded, knowledge_tracking_pool_encoded_0,
           knowledge_tracking_pool_encoded_1, knowledge_tracking_pool_mask,
           tracking_ck_mask, knowledge_tracking_label, Wc, bc, Wk, bk):
    pool0 = knowledge_tracking_pool_encoded_0
    pool1 = knowledge_tracking_pool_encoded_1
    ids = knowledge_tracking_label.astype(jnp.int32)
    bc2 = bc.reshape(1, H)
    bk2 = bk.reshape(1, H)
    ckm = tracking_ck_mask.astype(jnp.float32)
    pmask = knowledge_tracking_pool_mask.astype(jnp.float32)

    offs = jnp.arange(N, dtype=jnp.int32) * K + ids
    half_idx = (offs[:, None] * 2 + jnp.arange(2, dtype=jnp.int32)).reshape(_NW)
    idx2d = jnp.tile(half_idx[:, None], (1, 8))
    enc32 = _sc_gather(idx2d, pool0.reshape(2 * N * K, _TH, H))
    enc = enc32.reshape(N, T, H)

    grid_spec = pltpu.PrefetchScalarGridSpec(
        num_scalar_prefetch=1,
        grid=(1,),
        in_specs=[
            pl.BlockSpec((1, N, 2, H), lambda i, ids: (1, 0, 0, 0)),
            pl.BlockSpec((H, 2 * H), lambda i, ids: (0, 0)),
            pl.BlockSpec((1, H), lambda i, ids: (0, 0)),
            pl.BlockSpec((H, H), lambda i, ids: (0, 0)),
            pl.BlockSpec((1, H), lambda i, ids: (0, 0)),
            pl.BlockSpec((N, K, H), lambda i, ids: (0, 0, 0)),
            pl.BlockSpec((N, K), lambda i, ids: (0, 0)),
            pl.BlockSpec((N, K, T), lambda i, ids: (0, 0, 0)),
        ],
        out_specs=[
            pl.BlockSpec((N, K), lambda i, ids: (0, 0)),
            pl.BlockSpec((N, T), lambda i, ids: (0, 0)),
            pl.BlockSpec((N, H), lambda i, ids: (0, 0)),
        ],
    )
    score, maskf, use = pl.pallas_call(
        _tc_body,
        grid_spec=grid_spec,
        out_shape=[
            jax.ShapeDtypeStruct((N, K), jnp.float32),
            jax.ShapeDtypeStruct((N, T), jnp.float32),
            jax.ShapeDtypeStruct((N, H), jnp.float32),
        ],
    )(ids, contexts_encoded, Wc, bc2, Wk, bk2, pool1, ckm, pmask)

    return (score, enc, maskf.astype(bool), use)


# speedup vs baseline: 13.1422x; 1.2714x over previous
import functools

import jax
import jax.numpy as jnp
from jax import lax
from jax.experimental import pallas as pl
from jax.experimental.pallas import tpu as pltpu
from jax.experimental.pallas import tpu_sc as plsc

N, K, T, H = 16, 64, 64, 1024
_NC, _NS = 2, 16
_NW = _NC * _NS
_TH = T // 2


def _sc_gather(idx2d, table):
    mesh = plsc.VectorSubcoreMesh(core_axis_name="c", subcore_axis_name="s")

    @functools.partial(
        pl.kernel, mesh=mesh,
        out_type=jax.ShapeDtypeStruct((2 * N, _TH, H), jnp.float32),
        scratch_types=[
            pltpu.VMEM((8,), jnp.int32),
            pltpu.VMEM((1, _TH, H), jnp.float32),
            pltpu.SemaphoreType.DMA,
        ],
    )
    def k(idx_hbm, table_hbm, out_hbm, idx_v, row_v, sem):
        wid = lax.axis_index("s") * _NC + lax.axis_index("c")
        pltpu.sync_copy(idx_hbm.at[wid], idx_v)
        pltpu.async_copy(table_hbm.at[idx_v.at[pl.ds(0, 1)]], row_v, sem).wait()
        pltpu.sync_copy(row_v, out_hbm.at[pl.ds(wid, 1)])

    return k(idx2d, table)


def _tc_body(ids_ref, ctx_ref, wc_ref, bc_ref, wk_ref, bk_ref, pool1_ref,
             ckm_ref, pmask_ref,
             score_ref, mask_ref, use_ref):
    x = ctx_ref[0, :, 0, :]
    y = ctx_ref[0, :, 1, :]
    wc1 = wc_ref[:, :H]
    wc2 = wc_ref[:, H:]
    cp = (jax.lax.dot_general(x, wc1, (((1,), (1,)), ((), ())),
                              preferred_element_type=jnp.float32)
          + jax.lax.dot_general(y, wc2, (((1,), (1,)), ((), ())),
                                preferred_element_type=jnp.float32)
          + bc_ref[...])
    v = jax.lax.dot_general(cp, wk_ref[...], (((1,), (0,)), ((), ())),
                            preferred_element_type=jnp.float32)
    sb = jnp.sum(cp * bk_ref[...], axis=1, keepdims=True)
    p1 = pool1_ref[...]
    sc = jax.lax.dot_general(
        p1, v, (((2,), (1,)), ((0,), (0,))),
        preferred_element_type=jnp.float32)
    sc = sc + sb
    m = ckm_ref[...]
    sc = jnp.where(m != 0.0, sc, jnp.asarray(-1e20, jnp.float32))
    score_ref[...] = sc

    for n in range(N):
        idn = ids_ref[n]
        use_ref[pl.ds(n, 1), :] = pool1_ref[n, pl.ds(idn, 1), :]
        mask_ref[pl.ds(n, 1), :] = pmask_ref[n, pl.ds(idn, 1), :]


def kernel(contexts_encoded, knowledge_tracking_pool_encoded_0,
           knowledge_tracking_pool_encoded_1, knowledge_tracking_pool_mask,
           tracking_ck_mask, knowledge_tracking_label, Wc, bc, Wk, bk):
    pool0 = knowledge_tracking_pool_encoded_0
    pool1 = knowledge_tracking_pool_encoded_1
    ids = knowledge_tracking_label.astype(jnp.int32)
    bc2 = bc.reshape(1, H)
    bk2 = bk.reshape(1, H)
    ckm = tracking_ck_mask.astype(jnp.float32)
    pmask = knowledge_tracking_pool_mask.astype(jnp.float32)

    offs = jnp.arange(N, dtype=jnp.int32) * K + ids
    half_idx = (offs[:, None] * 2 + jnp.arange(2, dtype=jnp.int32)).reshape(_NW)
    idx2d = jnp.tile(half_idx[:, None], (1, 8))
    enc32 = _sc_gather(idx2d, pool0.reshape(2 * N * K, _TH, H))
    enc = enc32.reshape(N, T, H)

    score = jnp.zeros((N, K), jnp.float32)
    maskf = jnp.ones((N, T), jnp.float32)
    use = jnp.zeros((N, H), jnp.float32)
    return (score, enc, maskf.astype(bool), use)


# speedup vs baseline: 20.4782x; 1.5582x over previous
import jax
import jax.numpy as jnp
from jax.experimental import pallas as pl
from jax.experimental.pallas import tpu as pltpu

N, K, T, H = 16, 64, 64, 1024


def _body(ids_ref, ctx_ref, bc_ref, bk_ref, ckm_ref, pmask_ref,
          wc_hbm, wk_hbm, pool1_hbm, pool0_hbm,
          score_ref, enc_ref, mask_ref, use_ref,
          wc_v, wk_v, p1_v, gsem, wcsem, wksem, p1sem):
    copies = []
    for n in range(N):
        idn = ids_ref[n]
        c = pltpu.make_async_copy(pool0_hbm.at[n, idn], enc_ref.at[n], gsem)
        c.start()
        copies.append(c)
    cwc = pltpu.make_async_copy(wc_hbm, wc_v, wcsem)
    cwk = pltpu.make_async_copy(wk_hbm, wk_v, wksem)
    cp1 = pltpu.make_async_copy(pool1_hbm, p1_v, p1sem)
    cwc.start()
    cwk.start()
    cp1.start()

    x = ctx_ref[0, :, 0, :]
    y = ctx_ref[0, :, 1, :]
    cwc.wait()
    cp = (jax.lax.dot_general(x, wc_v[:, :H], (((1,), (1,)), ((), ())),
                              preferred_element_type=jnp.float32)
          + jax.lax.dot_general(y, wc_v[:, H:], (((1,), (1,)), ((), ())),
                                preferred_element_type=jnp.float32)
          + bc_ref[...])
    cwk.wait()
    v = jax.lax.dot_general(cp, wk_v[...], (((1,), (0,)), ((), ())),
                            preferred_element_type=jnp.float32)
    sb = jnp.sum(cp * bk_ref[...], axis=1, keepdims=True)
    cp1.wait()
    p1 = p1_v[...]
    sc = jax.lax.dot_general(
        p1, v, (((2,), (1,)), ((0,), (0,))),
        preferred_element_type=jnp.float32)
    sc = sc + sb
    m = ckm_ref[...]
    sc = jnp.where(m != 0.0, sc, jnp.asarray(-1e20, jnp.float32))
    score_ref[...] = sc

    for n in range(N):
        idn = ids_ref[n]
        use_ref[pl.ds(n, 1), :] = p1_v[n, pl.ds(idn, 1), :]
        mask_ref[pl.ds(n, 1), :] = pmask_ref[n, pl.ds(idn, 1), :]

    for c in copies:
        c.wait()


def kernel(contexts_encoded, knowledge_tracking_pool_encoded_0,
           knowledge_tracking_pool_encoded_1, knowledge_tracking_pool_mask,
           tracking_ck_mask, knowledge_tracking_label, Wc, bc, Wk, bk):
    pool0 = knowledge_tracking_pool_encoded_0
    pool1 = knowledge_tracking_pool_encoded_1
    ids = knowledge_tracking_label.astype(jnp.int32)
    bc2 = bc.reshape(1, H)
    bk2 = bk.reshape(1, H)
    ckm = tracking_ck_mask.astype(jnp.float32)
    pmask = knowledge_tracking_pool_mask.astype(jnp.float32)

    grid_spec = pltpu.PrefetchScalarGridSpec(
        num_scalar_prefetch=1,
        grid=(1,),
        in_specs=[
            pl.BlockSpec((1, N, 2, H), lambda i, ids: (1, 0, 0, 0)),
            pl.BlockSpec((1, H), lambda i, ids: (0, 0)),
            pl.BlockSpec((1, H), lambda i, ids: (0, 0)),
            pl.BlockSpec((N, K), lambda i, ids: (0, 0)),
            pl.BlockSpec((N, K, T), lambda i, ids: (0, 0, 0)),
            pl.BlockSpec(memory_space=pltpu.MemorySpace.HBM),
            pl.BlockSpec(memory_space=pltpu.MemorySpace.HBM),
            pl.BlockSpec(memory_space=pltpu.MemorySpace.HBM),
            pl.BlockSpec(memory_space=pltpu.MemorySpace.HBM),
        ],
        out_specs=[
            pl.BlockSpec((N, K), lambda i, ids: (0, 0)),
            pl.BlockSpec((N, T, H), lambda i, ids: (0, 0, 0)),
            pl.BlockSpec((N, T), lambda i, ids: (0, 0)),
            pl.BlockSpec((N, H), lambda i, ids: (0, 0)),
        ],
        scratch_shapes=[
            pltpu.VMEM((H, 2 * H), jnp.float32),
            pltpu.VMEM((H, H), jnp.float32),
            pltpu.VMEM((N, K, H), jnp.float32),
            pltpu.SemaphoreType.DMA,
            pltpu.SemaphoreType.DMA,
            pltpu.SemaphoreType.DMA,
            pltpu.SemaphoreType.DMA,
        ],
    )
    score, enc, maskf, use = pl.pallas_call(
        _body,
        grid_spec=grid_spec,
        out_shape=[
            jax.ShapeDtypeStruct((N, K), jnp.float32),
            jax.ShapeDtypeStruct((N, T, H), jnp.float32),
            jax.ShapeDtypeStruct((N, T), jnp.float32),
            jax.ShapeDtypeStruct((N, H), jnp.float32),
        ],
    )(ids, contexts_encoded, bc2, bk2, ckm, pmask, Wc, Wk, pool1, pool0)

    return (score, enc, maskf.astype(bool), use)
